# baseline (speedup 1.0000x reference)
import jax
import jax.numpy as jnp
from jax import lax
from jax.experimental import pallas as pl
from jax.experimental.pallas import tpu as pltpu
from jax.experimental.pallas import tpu_sc as plsc

N = 10000
E = 160000
NC = 2
NS = 16
TILES = NC * NS
EPT = 5120
NBATCH = EPT // 128
EPAD = TILES * EPT
SINK = N
NROWS = 10240
ROWS_PER_SUB = NROWS // NS


def _sc_aggregate(nb: int, width: int, gather: bool):
    mesh = plsc.VectorSubcoreMesh(core_axis_name="c", subcore_axis_name="s",
                                  num_cores=NC, num_subcores=NS)
    scratch = [
        pltpu.VMEM((EPT,), jnp.int32),
        pltpu.VMEM((EPT + 128,), jnp.int32),
        pltpu.VMEM((EPT,), jnp.int32),
        pltpu.VMEM((128, width), jnp.float32),
        pltpu.VMEM((128, width), jnp.float32),
        pltpu.VMEM_SHARED((NROWS, width), jnp.float32),
        pltpu.SemaphoreType.DMA,
        pltpu.SemaphoreType.DMA,
    ]

    def body(*refs):
        if gather:
            (tab_hbm, src_hbm, dst_hbm, out_hbm,
             src_v, srcb_v, dst_v, buf0, buf1, acc, sem0, sem1) = refs
        else:
            (dst_hbm, out_hbm,
             src_v, srcb_v, dst_v, buf0, buf1, acc, sem0, sem1) = refs
        c = lax.axis_index("c")
        s = lax.axis_index("s")
        wid = c * NS + s

        pltpu.sync_copy(dst_hbm.at[pl.ds(wid * EPT, EPT)], dst_v)
        if gather:
            pltpu.sync_copy(src_hbm.at[pl.ds(wid * EPT, EPT)], src_v)

        def fill_buf(ref, val):
            vec = jnp.full((16,), val, jnp.float32)

            def fill_row(i, carry):
                for j in range(width // 16):
                    ref[i, pl.ds(j * 16, 16)] = vec
                return carry
            lax.fori_loop(0, 128, fill_row, 0)

        if not gather:
            fill_buf(buf1, 1.0)

        def zero_my_rows():
            base = s * ROWS_PER_SUB
            for k in range(ROWS_PER_SUB // 128):
                pltpu.sync_copy(buf0.at[pl.ds(0, 128)],
                                acc.at[pl.ds(base + k * 128, 128)])

        def gidx(j):
            return srcb_v.at[pl.ds(j * 128, 128)]

        def didx(j):
            return dst_v.at[pl.ds(j * 128, 128)]

        for b in range(nb):
            if gather:
                boff = jnp.int32(b * N)

                def off_body(i, carry):
                    v = src_v[pl.ds(i * 16, 16)]
                    srcb_v[pl.ds(i * 16, 16)] = v + boff
                    return carry
                lax.fori_loop(0, EPT // 16, off_body, 0)
                pad = jnp.full((16,), b * N, jnp.int32)
                for i in range(8):
                    srcb_v[pl.ds(EPT + i * 16, 16)] = pad

            fill_buf(buf0, 0.0)
            zero_my_rows()
            plsc.subcore_barrier()

            if gather:
                pltpu.async_copy(tab_hbm.at[gidx(0)], buf0, sem0)

                def acc_body(g, carry):
                    j = g * 2
                    pltpu.async_copy(tab_hbm.at[gidx(j + 1)], buf1, sem1)
                    pltpu.make_async_copy(tab_hbm.at[gidx(j)],
                                          buf0, sem0).wait()
                    pltpu.sync_copy(buf0, acc.at[didx(j)], add=True)
                    pltpu.async_copy(tab_hbm.at[gidx(j + 2)], buf0, sem0)
                    pltpu.make_async_copy(tab_hbm.at[gidx(j + 1)],
                                          buf1, sem1).wait()
                    pltpu.sync_copy(buf1, acc.at[didx(j + 1)], add=True)
                    return carry
                lax.fori_loop(0, NBATCH // 2, acc_body, 0)
                pltpu.make_async_copy(tab_hbm.at[gidx(NBATCH)],
                                      buf0, sem0).wait()
            else:
                def acc_body1(j, carry):
                    pltpu.sync_copy(buf1, acc.at[didx(j)], add=True)
                    return carry
                lax.fori_loop(0, NBATCH, acc_body1, 0)

            plsc.subcore_barrier()
            base = s * ROWS_PER_SUB
            pltpu.sync_copy(acc.at[pl.ds(base, ROWS_PER_SUB)],
                            out_hbm.at[c, b, pl.ds(base, ROWS_PER_SUB)])

    out_type = jax.ShapeDtypeStruct((NC, nb, NROWS, width), jnp.float32)
    return pl.kernel(body, out_type=out_type, mesh=mesh,
                     scratch_types=scratch)


def _tc_dinv(d):
    BM = 1000

    def body(d_ref, o_ref):
        deg = d_ref[0, 0, :, 0:1] + d_ref[1, 0, :, 0:1] + 1.0
        o_ref[...] = lax.rsqrt(deg)

    return pl.pallas_call(
        body,
        grid=(N // BM,),
        in_specs=[pl.BlockSpec((2, 1, BM, 16), lambda m: (0, 0, m, 0))],
        out_specs=pl.BlockSpec((BM, 1), lambda m: (m, 0)),
        out_shape=jax.ShapeDtypeStruct((N, 1), jnp.float32),
    )(d)


def _tc_first(x, w, dinv, nbout):
    BM = 1000

    def body(x_ref, w_ref, di_ref, o_ref):
        p = jnp.dot(x_ref[...], w_ref[...], preferred_element_type=jnp.float32)
        o_ref[0] = di_ref[...] * p

    kin = x.shape[1]
    return pl.pallas_call(
        body,
        grid=(nbout, N // BM),
        in_specs=[
            pl.BlockSpec((BM, kin), lambda b, m: (m, 0)),
            pl.BlockSpec((kin, 128), lambda b, m: (0, b)),
            pl.BlockSpec((BM, 1), lambda b, m: (m, 0)),
        ],
        out_specs=pl.BlockSpec((1, BM, 128), lambda b, m: (b, m, 0)),
        out_shape=jax.ShapeDtypeStruct((nbout, N, 128), jnp.float32),
    )(x, w, dinv)


def _tc_layer(s, pt, dinv, bias2d, w, nbin, nbout):
    BM = 1000

    def body(s_ref, pt_ref, di_ref, b_ref, w_ref, o_ref):
        k = pl.program_id(2)
        di = di_ref[...]
        xin = di * (s_ref[0, 0] + s_ref[1, 0] + pt_ref[0]) + b_ref[...]
        xin = jnp.maximum(xin, 0.0)
        part = jnp.dot(xin, w_ref[...], preferred_element_type=jnp.float32)

        @pl.when(k == 0)
        def _():
            o_ref[0] = part

        @pl.when(k > 0)
        def _():
            o_ref[0] += part

        @pl.when(k == nbin - 1)
        def _():
            o_ref[0] = di * o_ref[0]

    return pl.pallas_call(
        body,
        grid=(nbout, N // BM, nbin),
        in_specs=[
            pl.BlockSpec((2, 1, BM, 128), lambda b, m, k: (0, k, m, 0)),
            pl.BlockSpec((1, BM, 128), lambda b, m, k: (k, m, 0)),
            pl.BlockSpec((BM, 1), lambda b, m, k: (m, 0)),
            pl.BlockSpec((1, 128), lambda b, m, k: (0, k)),
            pl.BlockSpec((128, 128), lambda b, m, k: (k, b)),
        ],
        out_specs=pl.BlockSpec((1, BM, 128), lambda b, m, k: (b, m, 0)),
        out_shape=jax.ShapeDtypeStruct((nbout, N, 128), jnp.float32),
    )(s, pt, dinv, bias2d, w)


def _tc_head(s, pt, dinv, b4_2d, batch2d, lin1_W, lin1_b2d, lin_W, lin_b2d):
    NG = 64

    def body(s_ref, pt_ref, di_ref, b4_ref, bt_ref,
             w1_ref, bb1_ref, w2_ref, bb2_ref, o_ref):
        h4 = di_ref[...] * (s_ref[0, 0, :N] + s_ref[1, 0, :N] + pt_ref[...])
        h4 = h4[:, :64] + b4_ref[...]
        gids = lax.broadcasted_iota(jnp.int32, (NG, N), 0)
        oh = (gids == bt_ref[...]).astype(jnp.float32)
        pool = jnp.dot(oh, h4, preferred_element_type=jnp.float32)
        cnt = jnp.sum(oh, axis=1, keepdims=True)
        mean = pool / jnp.maximum(cnt, 1.0)
        g = jnp.maximum(
            jnp.dot(mean, w1_ref[...], preferred_element_type=jnp.float32)
            + bb1_ref[...], 0.0)
        o_ref[...] = (jnp.dot(g, w2_ref[...],
                              preferred_element_type=jnp.float32)
                      + bb2_ref[...])

    return pl.pallas_call(
        body,
        out_shape=jax.ShapeDtypeStruct((NG, 2), jnp.float32),
    )(s, pt, dinv, b4_2d, batch2d, lin1_W, lin1_b2d, lin_W, lin_b2d)


def kernel(x, edge_index, batch, W1, b1, W2, b2, W3, b3, W4, b4,
           lin1_W, lin1_b, lin_W, lin_b):
    src = edge_index[0].astype(jnp.int32)
    dst = edge_index[1].astype(jnp.int32)
    npad = EPAD - E
    src_p = jnp.concatenate([src, jnp.zeros((npad,), jnp.int32)])
    dst_p = jnp.concatenate([dst, jnp.full((npad,), SINK, jnp.int32)])

    deg = _sc_aggregate(1, 16, gather=False)(dst_p)
    dinv = _tc_dinv(deg)

    pt1 = _tc_first(x, W1, dinv, 4)
    s1 = _sc_aggregate(4, 128, True)(pt1.reshape(4 * N, 128), src_p, dst_p)
    pt2 = _tc_layer(s1, pt1, dinv, b1.reshape(1, 512), W2, 4, 2)
    s2 = _sc_aggregate(2, 128, True)(pt2.reshape(2 * N, 128), src_p, dst_p)
    pt3 = _tc_layer(s2, pt2, dinv, b2.reshape(1, 256), W3, 2, 1)
    s3 = _sc_aggregate(1, 128, True)(pt3.reshape(1 * N, 128), src_p, dst_p)
    W4p = jnp.pad(W4, ((0, 0), (0, 64)))
    pt4 = _tc_layer(s3, pt3, dinv, b3.reshape(1, 128), W4p, 1, 1)
    s4 = _sc_aggregate(1, 128, True)(pt4.reshape(1 * N, 128), src_p, dst_p)

    out = _tc_head(s4, pt4.reshape(N, 128), dinv, b4.reshape(1, 64),
                   batch.astype(jnp.int32).reshape(1, N),
                   lin1_W, lin1_b.reshape(1, 32), lin_W, lin_b.reshape(1, 2))
    return out


# speedup vs baseline: 1.5314x; 1.5314x over previous
import jax
import jax.numpy as jnp
from jax import lax
from jax.experimental import pallas as pl
from jax.experimental.pallas import tpu as pltpu
from jax.experimental.pallas import tpu_sc as plsc

N = 10000
E = 160000
NC = 2
NS = 16
TILES = NC * NS
EPT = 5120
NBATCH = EPT // 128
EPAD = TILES * EPT
SINK = N
NROWS = 10240
ROWS_PER_SUB = NROWS // NS
CB2 = 256
RPS = NROWS // TILES


def _sc_aggregate(nb: int, width: int, gather: bool):
    mesh = plsc.VectorSubcoreMesh(core_axis_name="c", subcore_axis_name="s",
                                  num_cores=NC, num_subcores=NS)
    scratch = [
        pltpu.VMEM((EPT,), jnp.int32),
        pltpu.VMEM((EPT + 128,), jnp.int32),
        pltpu.VMEM((EPT,), jnp.int32),
        pltpu.VMEM((128, width), jnp.float32),
        pltpu.VMEM((128, width), jnp.float32),
        pltpu.VMEM_SHARED((NROWS, width), jnp.float32),
        pltpu.SemaphoreType.DMA,
        pltpu.SemaphoreType.DMA,
    ]

    def body(*refs):
        if gather:
            (tab_hbm, src_hbm, dst_hbm, out_hbm,
             src_v, srcb_v, dst_v, buf0, buf1, acc, sem0, sem1) = refs
        else:
            (dst_hbm, out_hbm,
             src_v, srcb_v, dst_v, buf0, buf1, acc, sem0, sem1) = refs
        c = lax.axis_index("c")
        s = lax.axis_index("s")
        wid = c * NS + s

        pltpu.sync_copy(dst_hbm.at[pl.ds(wid * EPT, EPT)], dst_v)
        if gather:
            pltpu.sync_copy(src_hbm.at[pl.ds(wid * EPT, EPT)], src_v)

        def fill_buf(ref, val):
            vec = jnp.full((16,), val, jnp.float32)

            def fill_row(i, carry):
                for j in range(width // 16):
                    ref[i, pl.ds(j * 16, 16)] = vec
                return carry
            lax.fori_loop(0, 128, fill_row, 0)

        if not gather:
            fill_buf(buf1, 1.0)

        def zero_my_rows():
            base = s * ROWS_PER_SUB
            for k in range(ROWS_PER_SUB // 128):
                pltpu.sync_copy(buf0.at[pl.ds(0, 128)],
                                acc.at[pl.ds(base + k * 128, 128)])

        def gidx(j):
            return srcb_v.at[pl.ds(j * 128, 128)]

        def didx(j):
            return dst_v.at[pl.ds(j * 128, 128)]

        for b in range(nb):
            if gather:
                boff = jnp.int32(b * N)

                def off_body(i, carry):
                    v = src_v[pl.ds(i * 16, 16)]
                    srcb_v[pl.ds(i * 16, 16)] = v + boff
                    return carry
                lax.fori_loop(0, EPT // 16, off_body, 0)
                pad = jnp.full((16,), b * N, jnp.int32)
                for i in range(8):
                    srcb_v[pl.ds(EPT + i * 16, 16)] = pad

            fill_buf(buf0, 0.0)
            zero_my_rows()
            plsc.subcore_barrier()

            if gather:
                pltpu.async_copy(tab_hbm.at[gidx(0)], buf0, sem0)

                def acc_body(g, carry):
                    j = g * 2
                    pltpu.async_copy(tab_hbm.at[gidx(j + 1)], buf1, sem1)
                    pltpu.make_async_copy(tab_hbm.at[gidx(j)],
                                          buf0, sem0).wait()
                    pltpu.sync_copy(buf0, acc.at[didx(j)], add=True)
                    pltpu.async_copy(tab_hbm.at[gidx(j + 2)], buf0, sem0)
                    pltpu.make_async_copy(tab_hbm.at[gidx(j + 1)],
                                          buf1, sem1).wait()
                    pltpu.sync_copy(buf1, acc.at[didx(j + 1)], add=True)
                    return carry
                lax.fori_loop(0, NBATCH // 2, acc_body, 0)
                pltpu.make_async_copy(tab_hbm.at[gidx(NBATCH)],
                                      buf0, sem0).wait()
            else:
                def acc_body1(j, carry):
                    pltpu.sync_copy(buf1, acc.at[didx(j)], add=True)
                    return carry
                lax.fori_loop(0, NBATCH, acc_body1, 0)

            plsc.subcore_barrier()
            base = s * ROWS_PER_SUB
            pltpu.sync_copy(acc.at[pl.ds(base, ROWS_PER_SUB)],
                            out_hbm.at[c, b, pl.ds(base, ROWS_PER_SUB)])

    out_type = jax.ShapeDtypeStruct((NC, nb, NROWS, width), jnp.float32)
    return pl.kernel(body, out_type=out_type, mesh=mesh,
                     scratch_types=scratch)


def _sc_gather_agg(nb: int):
    mesh = plsc.VectorSubcoreMesh(core_axis_name="c", subcore_axis_name="s",
                                  num_cores=NC, num_subcores=NS)
    scratch = [
        pltpu.VMEM((48,), jnp.int32),
        pltpu.VMEM((CB2,), jnp.int32),
        pltpu.VMEM((CB2,), jnp.int32),
        pltpu.VMEM((CB2,), jnp.int32),
        pltpu.VMEM((CB2,), jnp.int32),
        pltpu.VMEM((CB2,), jnp.int32),
        pltpu.VMEM((CB2,), jnp.int32),
        pltpu.VMEM((CB2, 128), jnp.float32),
        pltpu.VMEM((CB2, 128), jnp.float32),
        pltpu.VMEM((RPS * 128,), jnp.float32),
        pltpu.SemaphoreType.DMA,
        pltpu.SemaphoreType.DMA,
    ]

    def body(tab, srcr, dstr, bndr, zer, out,
             bbuf, sA, sB, dA, dB, iA, iB, gA, gB, acc, gsA, gsB):
        c = lax.axis_index("c")
        s = lax.axis_index("s")
        t = c * NS + s
        iota = lax.iota(jnp.int32, 16)
        cvs = [iota + j * 16 for j in range(8)]

        pltpu.sync_copy(bndr, bbuf)

        def bext(i):
            q = i // 16
            r = i % 16
            v = bbuf[pl.ds(q * 16, 16)]
            return jnp.sum(jnp.where(iota == r, v, 0))

        lo = bext(t)
        hi = bext(t + 1)
        lo8 = (lo // 8) * 8
        head = lo - lo8
        nstream = hi - lo8
        ntrip = (nstream + CB2 - 1) // CB2
        base_row = t * RPS

        def stage(j, sb, db, ib, gb, sem, boff):
            off = lo8 + j * CB2
            pltpu.sync_copy(srcr.at[pl.ds(off, CB2)], sb)
            pltpu.sync_copy(dstr.at[pl.ds(off, CB2)], db)

            def cb_(k, carry):
                v = sb[pl.ds(k * 16, 16)]
                ib[pl.ds(k * 16, 16)] = v + boff
                return carry
            lax.fori_loop(0, CB2 // 16, cb_, 0)
            pltpu.async_copy(tab.at[ib], gb, sem)

        def accum(j, db, gb):
            start = jnp.maximum(head - j * CB2, 0)
            cnt = jnp.minimum(CB2, nstream - j * CB2)

            def edge(e, carry):
                dv = plsc.load_gather(db, [jnp.full((16,), 0, jnp.int32) + e])
                rowoff = (dv - base_row) * 128
                for jj in range(8):
                    vals = gb[e, pl.ds(jj * 16, 16)]
                    plsc.addupdate_scatter(acc, [rowoff + cvs[jj]], vals)
                return carry
            lax.fori_loop(start, cnt, edge, 0)

        for b in range(nb):
            boff = b * N
            pltpu.sync_copy(zer, acc)

            @pl.when(ntrip > 0)
            def _():
                stage(0, sA, dA, iA, gA, gsA, boff)

            def step(k, carry):
                @pl.when(lax.rem(k, 2) == 0)
                def _():
                    @pl.when(k + 1 < ntrip)
                    def _():
                        stage(k + 1, sB, dB, iB, gB, gsB, boff)
                    pltpu.make_async_copy(tab.at[iA], gA, gsA).wait()
                    accum(k, dA, gA)

                @pl.when(lax.rem(k, 2) == 1)
                def _():
                    @pl.when(k + 1 < ntrip)
                    def _():
                        stage(k + 1, sA, dA, iA, gA, gsA, boff)
                    pltpu.make_async_copy(tab.at[iB], gB, gsB).wait()
                    accum(k, dB, gB)
                return carry
            lax.fori_loop(0, ntrip, step, 0)

            pltpu.sync_copy(acc, out.at[b, pl.ds(base_row * 128, RPS * 128)])

    out_type = jax.ShapeDtypeStruct((nb, NROWS * 128), jnp.float32)
    return pl.kernel(body, out_type=out_type, mesh=mesh,
                     scratch_types=scratch,
                     compiler_params=pltpu.CompilerParams(
                         needs_layout_passes=False))


def _tc_dinv(d):
    BM = 1000

    def body(d_ref, o_ref):
        deg = d_ref[0, 0, :, 0:1] + d_ref[1, 0, :, 0:1] + 1.0
        o_ref[...] = lax.rsqrt(deg)

    return pl.pallas_call(
        body,
        grid=(N // BM,),
        in_specs=[pl.BlockSpec((2, 1, BM, 16), lambda m: (0, 0, m, 0))],
        out_specs=pl.BlockSpec((BM, 1), lambda m: (m, 0)),
        out_shape=jax.ShapeDtypeStruct((N, 1), jnp.float32),
    )(d)


def _tc_first(x, w, dinv, nbout):
    BM = 1000

    def body(x_ref, w_ref, di_ref, o_ref):
        p = jnp.dot(x_ref[...], w_ref[...], preferred_element_type=jnp.float32)
        o_ref[0] = di_ref[...] * p

    kin = x.shape[1]
    return pl.pallas_call(
        body,
        grid=(nbout, N // BM),
        in_specs=[
            pl.BlockSpec((BM, kin), lambda b, m: (m, 0)),
            pl.BlockSpec((kin, 128), lambda b, m: (0, b)),
            pl.BlockSpec((BM, 1), lambda b, m: (m, 0)),
        ],
        out_specs=pl.BlockSpec((1, BM, 128), lambda b, m: (b, m, 0)),
        out_shape=jax.ShapeDtypeStruct((nbout, N, 128), jnp.float32),
    )(x, w, dinv)


def _tc_layer(s, pt, dinv, bias2d, w, nbin, nbout):
    BM = 1000

    def body(s_ref, pt_ref, di_ref, b_ref, w_ref, o_ref):
        k = pl.program_id(2)
        di = di_ref[...]
        xin = di * (s_ref[0] + pt_ref[0]) + b_ref[...]
        xin = jnp.maximum(xin, 0.0)
        part = jnp.dot(xin, w_ref[...], preferred_element_type=jnp.float32)

        @pl.when(k == 0)
        def _():
            o_ref[0] = part

        @pl.when(k > 0)
        def _():
            o_ref[0] += part

        @pl.when(k == nbin - 1)
        def _():
            o_ref[0] = di * o_ref[0]

    return pl.pallas_call(
        body,
        grid=(nbout, N // BM, nbin),
        in_specs=[
            pl.BlockSpec((1, BM, 128), lambda b, m, k: (k, m, 0)),
            pl.BlockSpec((1, BM, 128), lambda b, m, k: (k, m, 0)),
            pl.BlockSpec((BM, 1), lambda b, m, k: (m, 0)),
            pl.BlockSpec((1, 128), lambda b, m, k: (0, k)),
            pl.BlockSpec((128, 128), lambda b, m, k: (k, b)),
        ],
        out_specs=pl.BlockSpec((1, BM, 128), lambda b, m, k: (b, m, 0)),
        out_shape=jax.ShapeDtypeStruct((nbout, N, 128), jnp.float32),
    )(s, pt, dinv, bias2d, w)


def _tc_head(s, pt, dinv, b4_2d, batch2d, lin1_W, lin1_b2d, lin_W, lin_b2d):
    NG = 64

    def body(s_ref, pt_ref, di_ref, b4_ref, bt_ref,
             w1_ref, bb1_ref, w2_ref, bb2_ref, o_ref):
        h4 = di_ref[...] * (s_ref[0, :N] + pt_ref[...])
        h4 = h4[:, :64] + b4_ref[...]
        gids = lax.broadcasted_iota(jnp.int32, (NG, N), 0)
        oh = (gids == bt_ref[...]).astype(jnp.float32)
        pool = jnp.dot(oh, h4, preferred_element_type=jnp.float32)
        cnt = jnp.sum(oh, axis=1, keepdims=True)
        mean = pool / jnp.maximum(cnt, 1.0)
        g = jnp.maximum(
            jnp.dot(mean, w1_ref[...], preferred_element_type=jnp.float32)
            + bb1_ref[...], 0.0)
        o_ref[...] = (jnp.dot(g, w2_ref[...],
                              preferred_element_type=jnp.float32)
                      + bb2_ref[...])

    return pl.pallas_call(
        body,
        out_shape=jax.ShapeDtypeStruct((NG, 2), jnp.float32),
    )(s, pt, dinv, b4_2d, batch2d, lin1_W, lin1_b2d, lin_W, lin_b2d)


def kernel(x, edge_index, batch, W1, b1, W2, b2, W3, b3, W4, b4,
           lin1_W, lin1_b, lin_W, lin_b):
    src = edge_index[0].astype(jnp.int32)
    dst = edge_index[1].astype(jnp.int32)
    npad = EPAD - E
    dst_p = jnp.concatenate([dst, jnp.full((npad,), SINK, jnp.int32)])
    dst_s, src_s = lax.sort_key_val(dst, src)
    bnd = jnp.searchsorted(
        dst_s, jnp.arange(33, dtype=jnp.int32) * RPS).astype(jnp.int32)
    bnd48 = jnp.concatenate([bnd, jnp.zeros((15,), jnp.int32)])
    src_sp = jnp.concatenate([src_s, jnp.zeros((CB2,), jnp.int32)])
    dst_sp = jnp.concatenate([dst_s, jnp.full((CB2,), NROWS - 1, jnp.int32)])

    deg = _sc_aggregate(1, 16, gather=False)(dst_p)
    dinv = _tc_dinv(deg)

    zer = jnp.zeros((RPS * 128,), jnp.float32)

    def agg(pt, nb):
        out = _sc_gather_agg(nb)(pt.reshape(nb * N, 128),
                                 src_sp, dst_sp, bnd48, zer)
        return out.reshape(nb, NROWS, 128)

    pt1 = _tc_first(x, W1, dinv, 4)
    s1 = agg(pt1, 4)
    pt2 = _tc_layer(s1, pt1, dinv, b1.reshape(1, 512), W2, 4, 2)
    s2 = agg(pt2, 2)
    pt3 = _tc_layer(s2, pt2, dinv, b2.reshape(1, 256), W3, 2, 1)
    s3 = agg(pt3, 1)
    W4p = jnp.pad(W4, ((0, 0), (0, 64)))
    pt4 = _tc_layer(s3, pt3, dinv, b3.reshape(1, 128), W4p, 1, 1)
    s4 = agg(pt4, 1)

    out = _tc_head(s4, pt4.reshape(N, 128), dinv, b4.reshape(1, 64),
                   batch.astype(jnp.int32).reshape(1, N),
                   lin1_W, lin1_b.reshape(1, 32), lin_W, lin_b.reshape(1, 2))
    return out
